# baseline (speedup 1.0000x reference)
import jax
import jax.numpy as jnp
from jax.experimental import pallas as pl
from jax.experimental.pallas import tpu as pltpu

B, C, H, W = 8, 384, 14, 14
T = B * H * W
TQ = 2048
E, K, R = 8, 2, 4
DH = R * C
CAP = int(1.25 * T * K / E)
NCAP = 512
NQ = E * NCAP


def _f_body(x_ref, w_ref, dwb_ref, lnw_ref, lnb_ref, rw_ref,
            xpad_ref, res_ref, pri_ref, i1_ref, i2_ref, w1n_ref, w2n_ref,
            a_ref, pad_ref):
    pad_ref[...] = jnp.zeros((H + 6, W + 6, B, C), jnp.float32)
    pad_ref[3:3 + H, 3:3 + W, :, :] = x_ref[...]
    acc = jnp.zeros((H, W, B, C), jnp.float32)
    for dh in range(7):
        for dw in range(7):
            wv = w_ref[dh * 7 + dw, :]
            acc = acc + pad_ref[dh:dh + H, dw:dw + W, :, :] * wv
    acc = acc + dwb_ref[...]
    mu = jnp.mean(acc, axis=-1, keepdims=True)
    xc = acc - mu
    var = jnp.mean(xc * xc, axis=-1, keepdims=True)
    xln = xc / jnp.sqrt(var + 1e-6) * lnw_ref[...] + lnb_ref[...]
    xs = xln.reshape(T, C)
    xpad_ref[...] = jnp.zeros((TQ, C), jnp.float32)
    xpad_ref[0:T, :] = xs
    res_ref[...] = jnp.zeros((TQ, C), jnp.float32)
    res_ref[0:T, :] = xs + x_ref[...].reshape(T, C)
    logits = jnp.dot(xs, rw_ref[...], preferred_element_type=jnp.float32)
    lane = jax.lax.broadcasted_iota(jnp.int32, (T, E), 1)
    m = jnp.max(logits, axis=-1, keepdims=True)
    p = jnp.exp(logits - m)
    probs = p / jnp.sum(p, axis=-1, keepdims=True)
    m1 = jnp.max(probs, axis=-1, keepdims=True)
    i1 = jnp.min(jnp.where(probs == m1, lane, E), axis=-1, keepdims=True)
    pm = jnp.where(lane == i1, -1.0, probs)
    m2 = jnp.max(pm, axis=-1, keepdims=True)
    i2 = jnp.min(jnp.where(pm == m2, lane, E), axis=-1, keepdims=True)
    s = m1 + m2
    pri_ref[...] = jnp.full((TQ, 1), -1.0, jnp.float32)
    pri_ref[0:T, :] = m1
    i1_ref[...] = jnp.zeros((TQ, 1), jnp.int32)
    i1_ref[0:T, :] = i1
    i2_ref[...] = jnp.zeros((TQ, 1), jnp.int32)
    i2_ref[0:T, :] = i2
    w1n_ref[...] = jnp.zeros((TQ, 1), jnp.float32)
    w1n_ref[0:T, :] = m1 / s
    w2n_ref[...] = jnp.zeros((TQ, 1), jnp.float32)
    w2n_ref[0:T, :] = m2 / s
    a_ref[...] = jnp.zeros((TQ, E), jnp.float32)
    a_ref[0:T, :] = ((lane == i1) | (lane == i2)).astype(jnp.float32)


def _r_body(pc_ref, pr_ref, a_ref, i1_ref, i2_ref, w1n_ref, w2n_ref,
            pos1_ref, pos2_ref, g1_ref, g2_ref):
    prow = pr_ref[...]
    amat = a_ref[...]
    cparts = []
    for rb in range(TQ // 128):
        r0 = rb * 128
        pcol = pc_ref[r0:r0 + 128, :]
        tcol = jax.lax.broadcasted_iota(jnp.int32, (128, TQ), 1)
        trow = jax.lax.broadcasted_iota(jnp.int32, (128, TQ), 0) + r0
        gt = prow > pcol
        eq = (prow == pcol) & (tcol < trow)
        mblk = jnp.where(gt | eq, 1.0, 0.0)
        cparts.append(jnp.dot(mblk, amat,
                              preferred_element_type=jnp.float32))
    cnt = jnp.concatenate(cparts, axis=0)
    lane = jax.lax.broadcasted_iota(jnp.int32, (TQ, E), 1)
    i1 = i1_ref[...]
    i2 = i2_ref[...]
    r1 = jnp.sum(jnp.where(lane == i1, cnt, 0.0), axis=-1, keepdims=True)
    r2 = jnp.sum(jnp.where(lane == i2, cnt, 0.0), axis=-1, keepdims=True)
    vrow = jax.lax.broadcasted_iota(jnp.int32, (TQ, 1), 0) < T
    k1 = (r1 < CAP) & vrow
    k2 = (r2 < CAP) & vrow
    g1_ref[...] = w1n_ref[...] * k1.astype(jnp.float32)
    g2_ref[...] = w2n_ref[...] * k2.astype(jnp.float32)
    pos1_ref[...] = jnp.where(k1, i1 * NCAP + r1.astype(jnp.int32), NQ - 1)
    pos2_ref[...] = jnp.where(k2, i2 * NCAP + r2.astype(jnp.int32), NQ - 1)


def _cd_body(p1_ref, p2_ref, g1_ref, g2_ref, x_ref, w1_ref, b1_ref, w2_ref,
             b2_ref, ls_ref, res_ref, o_ref):
    e = pl.program_id(0)

    @pl.when(e == 0)
    def _():
        o_ref[...] = res_ref[...]

    q = jax.lax.broadcasted_iota(jnp.int32, (TQ, NCAP), 1) + e * NCAP
    mq1 = p1_ref[...] == q
    mq2 = p2_ref[...] == q
    qt = (mq1 | mq2).astype(jnp.bfloat16)
    xbf = x_ref[...].astype(jnp.bfloat16)
    d = jax.lax.dot_general(qt, xbf, (((0,), (0,)), ((), ())),
                            preferred_element_type=jnp.float32)
    h = jnp.dot(d.astype(jnp.bfloat16), w1_ref[0].astype(jnp.bfloat16),
                preferred_element_type=jnp.float32) + b1_ref[0]
    g = 0.5 * h * (1.0 + jax.lax.erf(h * 0.7071067811865476))
    y = jnp.dot(g.astype(jnp.bfloat16), w2_ref[0].astype(jnp.bfloat16),
                preferred_element_type=jnp.float32) + b2_ref[0]
    y = (y * ls_ref[...]).astype(jnp.bfloat16)
    wc = (jnp.where(mq1, g1_ref[...], 0.0)
          + jnp.where(mq2, g2_ref[...], 0.0)).astype(jnp.bfloat16)
    o_ref[...] += jnp.dot(wc, y, preferred_element_type=jnp.float32)


def kernel(input, dw_w, dw_b, ln_w, ln_b, router_w, w1, b1, w2, b2,
           layer_scale):
    f32 = jnp.float32
    x_t = jnp.transpose(input, (2, 3, 0, 1))
    wconv = jnp.transpose(dw_w[:, 0], (1, 2, 0)).reshape(49, C)

    col_f = jax.ShapeDtypeStruct((TQ, 1), f32)
    col_i = jax.ShapeDtypeStruct((TQ, 1), jnp.int32)
    xpad, resid, pri, i1, i2, w1n, w2n, amat = pl.pallas_call(
        _f_body,
        out_shape=[jax.ShapeDtypeStruct((TQ, C), f32),
                   jax.ShapeDtypeStruct((TQ, C), f32),
                   col_f, col_i, col_i, col_f, col_f,
                   jax.ShapeDtypeStruct((TQ, E), f32)],
        scratch_shapes=[pltpu.VMEM((H + 6, W + 6, B, C), f32)],
    )(x_t, wconv, dw_b, ln_w, ln_b, router_w)

    pos1, pos2, g1, g2 = pl.pallas_call(
        _r_body,
        out_shape=[col_i, col_i, col_f, col_f],
    )(pri, pri.reshape(1, TQ), amat, i1, i2, w1n, w2n)

    out = (resid + g1 + g2 + pos1 + pos2)[:T].reshape(H, W, B, C)
    return jnp.transpose(out, (2, 3, 0, 1))
    out_s = pl.pallas_call(
        _cd_body,
        grid=(E,),
        in_specs=[
            pl.BlockSpec((TQ, 1), lambda e: (0, 0)),
            pl.BlockSpec((TQ, 1), lambda e: (0, 0)),
            pl.BlockSpec((TQ, 1), lambda e: (0, 0)),
            pl.BlockSpec((TQ, 1), lambda e: (0, 0)),
            pl.BlockSpec((TQ, C), lambda e: (0, 0)),
            pl.BlockSpec((1, C, DH), lambda e: (e, 0, 0)),
            pl.BlockSpec((1, 1, DH), lambda e: (e, 0, 0)),
            pl.BlockSpec((1, DH, C), lambda e: (e, 0, 0)),
            pl.BlockSpec((1, 1, C), lambda e: (e, 0, 0)),
            pl.BlockSpec((1, C), lambda e: (0, 0)),
            pl.BlockSpec((TQ, C), lambda e: (0, 0)),
        ],
        out_specs=pl.BlockSpec((TQ, C), lambda e: (0, 0)),
        out_shape=jax.ShapeDtypeStruct((TQ, C), f32),
    )(pos1, pos2, g1, g2, xpad, w1, b1.reshape(E, 1, DH), w2,
      b2.reshape(E, 1, C), layer_scale.reshape(1, C), resid)

    out = out_s[:T].reshape(H, W, B, C)
    return jnp.transpose(out, (2, 3, 0, 1))
